# speedup vs baseline: 31.9148x; 31.9148x over previous
import functools

import jax
import jax.numpy as jnp
from jax.experimental import pallas as pl
from jax.experimental.pallas import tpu as pltpu

_B = 64


def _pick_block_rows(n: int) -> int:
    for r in (4000, 3200, 2560, 2048, 2000, 1600, 1280, 1024, 800, 640, 512,
              400, 320, 256, 160, 128, 64, 32, 16, 8):
        if n % r == 0:
            return r
    return n


def _fused_body(nb, d, g_ref, x_ref, wt_ref, b_ref, t_ref, out_ref,
                m_ref, s_ref, w_ref):
    step = pl.program_id(0)

    @pl.when(step == 0)
    def _init():
        m_ref[...] = jnp.zeros_like(m_ref)
        s_ref[...] = jnp.zeros_like(s_ref)
        w_ref[...] = jnp.zeros_like(w_ref)

    x = x_ref[...]
    h = jnp.dot(x, wt_ref[...], preferred_element_type=jnp.float32)
    h = jnp.maximum(h + b_ref[...], 0.0)
    logits = h * t_ref[...]

    bm = jnp.max(logits, axis=0, keepdims=True)
    m_old = m_ref[...]
    m_new = jnp.maximum(m_old, bm)
    corr = jnp.exp(m_old - m_new)

    e = jnp.exp(logits - m_new)
    ew = jnp.concatenate([e, e * h], axis=1)

    g = g_ref[0]
    seg = jax.lax.broadcasted_iota(jnp.int32, (_B, g.shape[1]), 0)
    oh = (g == seg).astype(jnp.float32)
    contrib = jnp.dot(oh, ew, preferred_element_type=jnp.float32)

    m_ref[...] = m_new
    s_ref[...] = s_ref[...] * corr + contrib[:, :d]
    w_ref[...] = w_ref[...] * corr + contrib[:, d:]

    @pl.when(step == nb - 1)
    def _fin():
        s = s_ref[...]
        out_ref[...] = jnp.where(s > 0.0, w_ref[...] / s, 0.0)


def _run(x, g3, wt, b2, t2, interpret=False):
    n, d = x.shape
    r = _pick_block_rows(n)
    nb = n // r
    body = functools.partial(_fused_body, nb, d)
    return pl.pallas_call(
        body,
        grid=(nb,),
        in_specs=[
            pl.BlockSpec((1, 1, r), lambda i: (i, 0, 0)),
            pl.BlockSpec((r, d), lambda i: (i, 0)),
            pl.BlockSpec((d, d), lambda i: (0, 0)),
            pl.BlockSpec((1, d), lambda i: (0, 0)),
            pl.BlockSpec((1, d), lambda i: (0, 0)),
        ],
        out_specs=pl.BlockSpec((_B, d), lambda i: (0, 0)),
        out_shape=jax.ShapeDtypeStruct((_B, d), jnp.float32),
        scratch_shapes=[
            pltpu.VMEM((1, d), jnp.float32),
            pltpu.VMEM((_B, d), jnp.float32),
            pltpu.VMEM((_B, d), jnp.float32),
        ],
        compiler_params=pltpu.CompilerParams(
            dimension_semantics=("arbitrary",)),
        interpret=interpret,
    )(g3, x, wt, b2, t2)


def kernel(x, graph_idx, batch_size, W, b, t):
    n, d = x.shape
    r = _pick_block_rows(n)
    g3 = graph_idx.astype(jnp.int32).reshape(n // r, 1, r)
    wt = W.T
    b2 = b.reshape(1, d)
    t2 = t.reshape(1, d)
    out = _run(x, g3, wt, b2, t2)
    return out + jnp.zeros((), dtype=jnp.float32) * batch_size
